# Initial kernel scaffold; baseline (speedup 1.0000x reference)
import jax
import jax.numpy as jnp
from jax.experimental import pallas as pl


def kernel(x, pe_table):
    raise NotImplementedError("write your pallas kernel here")


# speedup vs baseline: 1.5729x; 1.5729x over previous
import jax
import jax.numpy as jnp
from jax.experimental import pallas as pl


def _body(x_ref, pe_ref, out_ref):
    pe_t = jnp.transpose(pe_ref[...], (1, 0))
    out_ref[...] = x_ref[...] + pe_t[None, :, :]


def kernel(x, pe_table):
    b, d, s = x.shape
    s_blk = 512
    grid = (s // s_blk, b)
    return pl.pallas_call(
        _body,
        grid=grid,
        in_specs=[
            pl.BlockSpec((1, d, s_blk), lambda si, bi: (bi, 0, si)),
            pl.BlockSpec((s_blk, d), lambda si, bi: (si, 0)),
        ],
        out_specs=pl.BlockSpec((1, d, s_blk), lambda si, bi: (bi, 0, si)),
        out_shape=jax.ShapeDtypeStruct((b, d, s), x.dtype),
    )(x, pe_table)


# speedup vs baseline: 1.7900x; 1.1380x over previous
import jax
import jax.numpy as jnp
from jax.experimental import pallas as pl


def _body(x_ref, pe_ref, out_ref):
    pe_t = jnp.transpose(pe_ref[...], (1, 0))
    out_ref[...] = x_ref[...] + pe_t[None, :, :]


def kernel(x, pe_table):
    b, d, s = x.shape
    s_blk = 1024
    grid = (s // s_blk, b)
    return pl.pallas_call(
        _body,
        grid=grid,
        in_specs=[
            pl.BlockSpec((1, d, s_blk), lambda si, bi: (bi, 0, si)),
            pl.BlockSpec((s_blk, d), lambda si, bi: (si, 0)),
        ],
        out_specs=pl.BlockSpec((1, d, s_blk), lambda si, bi: (bi, 0, si)),
        out_shape=jax.ShapeDtypeStruct((b, d, s), x.dtype),
    )(x, pe_table)


# speedup vs baseline: 1.8764x; 1.0483x over previous
import jax
import jax.numpy as jnp
from jax.experimental import pallas as pl


def _body(x_ref, pe_ref, out_ref):
    pe_t = jnp.transpose(pe_ref[...], (1, 0))
    out_ref[...] = x_ref[...] + pe_t[None, :, :]


def kernel(x, pe_table):
    b, d, s = x.shape
    s_blk = 2048
    grid = (s // s_blk, b)
    return pl.pallas_call(
        _body,
        grid=grid,
        in_specs=[
            pl.BlockSpec((1, d, s_blk), lambda si, bi: (bi, 0, si)),
            pl.BlockSpec((s_blk, d), lambda si, bi: (si, 0)),
        ],
        out_specs=pl.BlockSpec((1, d, s_blk), lambda si, bi: (bi, 0, si)),
        out_shape=jax.ShapeDtypeStruct((b, d, s), x.dtype),
    )(x, pe_table)


# speedup vs baseline: 1.9746x; 1.0523x over previous
import jax
import jax.numpy as jnp
from jax.experimental import pallas as pl


def _body(x_ref, pe_ref, out_ref):
    pe_t = jnp.transpose(pe_ref[...], (1, 0))
    out_ref[...] = x_ref[...] + pe_t[None, :, :]


def kernel(x, pe_table):
    b, d, s = x.shape
    s_blk = 512
    grid = (s // s_blk,)
    return pl.pallas_call(
        _body,
        grid=grid,
        in_specs=[
            pl.BlockSpec((b, d, s_blk), lambda si: (0, 0, si)),
            pl.BlockSpec((s_blk, d), lambda si: (si, 0)),
        ],
        out_specs=pl.BlockSpec((b, d, s_blk), lambda si: (0, 0, si)),
        out_shape=jax.ShapeDtypeStruct((b, d, s), x.dtype),
    )(x, pe_table)
